# baseline (speedup 1.0000x reference)
import functools

import jax
import jax.numpy as jnp
from jax import lax
from jax.experimental import pallas as pl
from jax.experimental.pallas import tpu as pltpu
from jax.experimental.pallas import tpu_sc as plsc

N = 10000
NP = 10240
E = 320000
D = 128

NC = 2
NS = 16
NW = NC * NS
K = 100
ROWS = E // K
RPT = ROWS // NW
DK = 80
DROWS = E // DK
DRPT = DROWS // NW
NPT = NP // NS
DEGW = 16

_SC_PARAMS = pltpu.CompilerParams(use_tc_tiling_on_sc=False)
_MESH = dict(core_axis_name="c", subcore_axis_name="s")


def _zero_2d(ref, nrows, ncols):
    def row(r, carry):
        for j in range(ncols // 16):
            ref[r, pl.ds(j * 16, 16)] = jnp.zeros((16,), jnp.float32)
        return carry
    lax.fori_loop(0, nrows, row, 0)


def _sc_agg_body(h_hbm, src_hbm, dst_hbm, part_hbm, srcbuf, dstbuf, rows0,
                 rows1, acc, gsem0, gsem1, ssem0, ssem1):
    bufs = (rows0, rows1)
    gsem = (gsem0, gsem1)
    ssem = (ssem0, ssem1)

    c = lax.axis_index("c")
    s = lax.axis_index("s")
    wid = c * NS + s

    _zero_2d(rows0, K, D)
    pltpu.async_copy(src_hbm.at[wid], srcbuf, gsem0)
    pltpu.async_copy(dst_hbm.at[wid], dstbuf, gsem1)
    for r in range(NPT // 80):
        pltpu.async_copy(rows0.at[pl.ds(0, 80)],
                         acc.at[pl.ds(s * NPT + r * 80, 80)], ssem0)
    pltpu.make_async_copy(src_hbm.at[wid], srcbuf, gsem0).wait()
    pltpu.make_async_copy(dst_hbm.at[wid], dstbuf, gsem1).wait()
    for r in range(NPT // 80):
        pltpu.make_async_copy(rows0.at[pl.ds(0, 80)],
                              acc.at[pl.ds(s * NPT, 80)], ssem0).wait()
    plsc.subcore_barrier()

    pltpu.async_copy(h_hbm.at[srcbuf.at[0]], bufs[0], gsem[0])

    def step(i, carry):
        for b in range(2):
            j = i * 2 + b
            ob = 1 - b
            pltpu.make_async_copy(h_hbm.at[srcbuf.at[j]], bufs[b],
                                  gsem[b]).wait()
            pltpu.async_copy(bufs[b], acc.at[dstbuf.at[j]], ssem[b], add=True)

            @pl.when(j >= 1)
            def _():
                pltpu.make_async_copy(bufs[ob], acc.at[dstbuf.at[0]],
                                      ssem[ob]).wait()

            @pl.when(j + 1 < RPT)
            def _():
                pltpu.async_copy(h_hbm.at[srcbuf.at[j + 1]], bufs[ob],
                                 gsem[ob])
        return carry
    lax.fori_loop(0, RPT // 2, step, 0)
    pltpu.make_async_copy(bufs[1], acc.at[dstbuf.at[0]], ssem[1]).wait()

    plsc.subcore_barrier()

    pltpu.sync_copy(acc.at[pl.ds(s * NPT, NPT)],
                    part_hbm.at[c, pl.ds(s * NPT, NPT)])


def _make_sc_agg():
    return pl.kernel(
        _sc_agg_body,
        out_type=jax.ShapeDtypeStruct((NC, NP, D), jnp.float32),
        mesh=plsc.VectorSubcoreMesh(**_MESH),
        scratch_types=(
            pltpu.VMEM((RPT, K), jnp.int32),
            pltpu.VMEM((RPT, K), jnp.int32),
            pltpu.VMEM((K, D), jnp.float32),
            pltpu.VMEM((K, D), jnp.float32),
            pltpu.VMEM_SHARED((NP, D), jnp.float32),
            pltpu.SemaphoreType.DMA,
            pltpu.SemaphoreType.DMA,
            pltpu.SemaphoreType.DMA,
            pltpu.SemaphoreType.DMA,
        ),
        compiler_params=_SC_PARAMS,
    )


def _sc_deg_body(dst_hbm, degp_hbm, dstbuf, onesb, degacc, sem):
    c = lax.axis_index("c")
    s = lax.axis_index("s")
    wid = c * NS + s

    pltpu.sync_copy(dst_hbm.at[wid], dstbuf)

    _zero_2d(onesb, DK, DEGW)
    for r in range(NPT // DK):
        pltpu.sync_copy(onesb, degacc.at[pl.ds(s * NPT + r * DK, DK)])

    def orow(r, carry):
        onesb[r, pl.ds(0, 16)] = jnp.ones((16,), jnp.float32)
        return carry
    lax.fori_loop(0, DK, orow, 0)
    plsc.subcore_barrier()

    def fire(j, carry):
        pltpu.async_copy(onesb, degacc.at[dstbuf.at[j]], sem, add=True)
        return carry
    lax.fori_loop(0, DRPT, fire, 0)

    def drain(j, carry):
        pltpu.make_async_copy(onesb, degacc.at[dstbuf.at[0]], sem).wait()
        return carry
    lax.fori_loop(0, DRPT, drain, 0)

    plsc.subcore_barrier()
    pltpu.sync_copy(degacc.at[pl.ds(s * NPT, NPT)],
                    degp_hbm.at[c, pl.ds(s * NPT, NPT)])


def _make_sc_deg():
    return pl.kernel(
        _sc_deg_body,
        out_type=jax.ShapeDtypeStruct((NC, NP, DEGW), jnp.float32),
        mesh=plsc.VectorSubcoreMesh(**_MESH),
        scratch_types=(
            pltpu.VMEM((DRPT, DK), jnp.int32),
            pltpu.VMEM((DK, DEGW), jnp.float32),
            pltpu.VMEM_SHARED((NP, DEGW), jnp.float32),
            pltpu.SemaphoreType.DMA,
        ),
        compiler_params=_SC_PARAMS,
    )


def _tc_block_body(residual, p_ref, degp_ref, h_ref, w_ref, b_ref, o_ref):
    deg = degp_ref[0, :, 0:1] + degp_ref[1, :, 0:1]
    agg = (p_ref[0] + p_ref[1]) / jnp.maximum(deg, 1.0)
    y = jnp.dot(agg, w_ref[...], preferred_element_type=jnp.float32)
    y = jnp.maximum(y + b_ref[...], 0.0)
    if residual:
        y = y + h_ref[...]
    o_ref[...] = y


def _make_tc_block(residual):
    BN = 1000
    return pl.pallas_call(
        functools.partial(_tc_block_body, residual),
        grid=(N // BN,),
        in_specs=[
            pl.BlockSpec((NC, BN, D), lambda i: (0, i, 0)),
            pl.BlockSpec((NC, BN, DEGW), lambda i: (0, i, 0)),
            pl.BlockSpec((BN, D), lambda i: (i, 0)),
            pl.BlockSpec((D, D), lambda i: (0, 0)),
            pl.BlockSpec((1, D), lambda i: (0, 0)),
        ],
        out_specs=pl.BlockSpec((BN, D), lambda i: (i, 0)),
        out_shape=jax.ShapeDtypeStruct((N, D), jnp.float32),
    )


def kernel(h, W1, b1, W2, b2, W3, b3, edge_index):
    src = edge_index[0].astype(jnp.int32).reshape(NW, RPT, K)
    dst = edge_index[1].astype(jnp.int32).reshape(NW, RPT, K)
    dstd = edge_index[1].astype(jnp.int32).reshape(NW, DRPT, DK)

    deg_k = _make_sc_deg()
    agg = _make_sc_agg()
    blk_res = _make_tc_block(True)
    blk_last = _make_tc_block(False)

    degp = deg_k(dstd)

    part = agg(h, src, dst)
    h1 = blk_res(part, degp, h, W1, b1.reshape(1, D))
    part = agg(h1, src, dst)
    h2 = blk_res(part, degp, h1, W2, b2.reshape(1, D))
    part = agg(h2, src, dst)
    h3 = blk_last(part, degp, h2, W3, b3.reshape(1, D))
    return h3


# speedup vs baseline: 1.2155x; 1.2155x over previous
import functools

import jax
import jax.numpy as jnp
from jax import lax
from jax.experimental import pallas as pl
from jax.experimental.pallas import tpu as pltpu
from jax.experimental.pallas import tpu_sc as plsc

N = 10000
NP = 10240
E = 320000
D = 128

NC = 2
NS = 16
NW = NC * NS
K = 100
ROWS = E // K
RPT = ROWS // NW
DK = 80
DROWS = E // DK
DRPT = DROWS // NW
NPT = NP // NS
DEGW = 16

_SC_PARAMS = pltpu.CompilerParams(use_tc_tiling_on_sc=False)
_MESH = dict(core_axis_name="c", subcore_axis_name="s")


def _zero_2d(ref, nrows, ncols):
    def row(r, carry):
        for j in range(ncols // 16):
            ref[r, pl.ds(j * 16, 16)] = jnp.zeros((16,), jnp.float32)
        return carry
    lax.fori_loop(0, nrows, row, 0)


def _sc_agg_body(h_hbm, src_hbm, dst_hbm, part_hbm, srcbuf, dstbuf, rows0,
                 rows1, acc, gsem0, gsem1, ssem0, ssem1):
    bufs = (rows0, rows1)
    gsem = (gsem0, gsem1)
    ssem = (ssem0, ssem1)

    c = lax.axis_index("c")
    s = lax.axis_index("s")
    wid = c * NS + s

    _zero_2d(rows0, K, D)
    pltpu.async_copy(src_hbm.at[wid], srcbuf, gsem0)
    pltpu.async_copy(dst_hbm.at[wid], dstbuf, gsem1)
    for r in range(NPT // 80):
        pltpu.async_copy(rows0.at[pl.ds(0, 80)],
                         acc.at[pl.ds(s * NPT + r * 80, 80)], ssem0)
    pltpu.make_async_copy(src_hbm.at[wid], srcbuf, gsem0).wait()
    pltpu.make_async_copy(dst_hbm.at[wid], dstbuf, gsem1).wait()
    for r in range(NPT // 80):
        pltpu.make_async_copy(rows0.at[pl.ds(0, 80)],
                              acc.at[pl.ds(s * NPT, 80)], ssem0).wait()
    plsc.subcore_barrier()

    for b in range(2):
        pltpu.async_copy(h_hbm.at[srcbuf.at[b]], bufs[b], gsem[b])

    def step(i, carry):
        for b in range(2):
            j = i * 2 + b
            pltpu.make_async_copy(h_hbm.at[srcbuf.at[j]], bufs[b],
                                  gsem[b]).wait()
            pltpu.sync_copy(bufs[b], acc.at[dstbuf.at[j]], add=True)

            @pl.when(j + 2 < RPT)
            def _():
                pltpu.async_copy(h_hbm.at[srcbuf.at[j + 2]], bufs[b],
                                 gsem[b])
        return carry
    lax.fori_loop(0, RPT // 2, step, 0)

    plsc.subcore_barrier()

    pltpu.sync_copy(acc.at[pl.ds(s * NPT, NPT)],
                    part_hbm.at[c, pl.ds(s * NPT, NPT)])


def _make_sc_agg():
    return pl.kernel(
        _sc_agg_body,
        out_type=jax.ShapeDtypeStruct((NC, NP, D), jnp.float32),
        mesh=plsc.VectorSubcoreMesh(**_MESH),
        scratch_types=(
            pltpu.VMEM((RPT, K), jnp.int32),
            pltpu.VMEM((RPT, K), jnp.int32),
            pltpu.VMEM((K, D), jnp.float32),
            pltpu.VMEM((K, D), jnp.float32),
            pltpu.VMEM_SHARED((NP, D), jnp.float32),
            pltpu.SemaphoreType.DMA,
            pltpu.SemaphoreType.DMA,
            pltpu.SemaphoreType.DMA,
            pltpu.SemaphoreType.DMA,
        ),
        compiler_params=_SC_PARAMS,
    )


def _sc_deg_body(dst_hbm, degp_hbm, dstbuf, onesb, degacc, sem):
    c = lax.axis_index("c")
    s = lax.axis_index("s")
    wid = c * NS + s

    pltpu.sync_copy(dst_hbm.at[wid], dstbuf)

    _zero_2d(onesb, DK, DEGW)
    for r in range(NPT // DK):
        pltpu.sync_copy(onesb, degacc.at[pl.ds(s * NPT + r * DK, DK)])

    def orow(r, carry):
        onesb[r, pl.ds(0, 16)] = jnp.ones((16,), jnp.float32)
        return carry
    lax.fori_loop(0, DK, orow, 0)
    plsc.subcore_barrier()

    def fire(j, carry):
        pltpu.async_copy(onesb, degacc.at[dstbuf.at[j]], sem, add=True)
        return carry
    lax.fori_loop(0, DRPT, fire, 0)

    def drain(j, carry):
        pltpu.make_async_copy(onesb, degacc.at[dstbuf.at[0]], sem).wait()
        return carry
    lax.fori_loop(0, DRPT, drain, 0)

    plsc.subcore_barrier()
    pltpu.sync_copy(degacc.at[pl.ds(s * NPT, NPT)],
                    degp_hbm.at[c, pl.ds(s * NPT, NPT)])


def _make_sc_deg():
    return pl.kernel(
        _sc_deg_body,
        out_type=jax.ShapeDtypeStruct((NC, NP, DEGW), jnp.float32),
        mesh=plsc.VectorSubcoreMesh(**_MESH),
        scratch_types=(
            pltpu.VMEM((DRPT, DK), jnp.int32),
            pltpu.VMEM((DK, DEGW), jnp.float32),
            pltpu.VMEM_SHARED((NP, DEGW), jnp.float32),
            pltpu.SemaphoreType.DMA,
        ),
        compiler_params=_SC_PARAMS,
    )


def _tc_block_body(residual, p_ref, degp_ref, h_ref, w_ref, b_ref, o_ref):
    deg = degp_ref[0, :, 0:1] + degp_ref[1, :, 0:1]
    agg = (p_ref[0] + p_ref[1]) / jnp.maximum(deg, 1.0)
    y = jnp.dot(agg, w_ref[...], preferred_element_type=jnp.float32)
    y = jnp.maximum(y + b_ref[...], 0.0)
    if residual:
        y = y + h_ref[...]
    o_ref[...] = y


def _make_tc_block(residual):
    BN = 1000
    return pl.pallas_call(
        functools.partial(_tc_block_body, residual),
        grid=(N // BN,),
        in_specs=[
            pl.BlockSpec((NC, BN, D), lambda i: (0, i, 0)),
            pl.BlockSpec((NC, BN, DEGW), lambda i: (0, i, 0)),
            pl.BlockSpec((BN, D), lambda i: (i, 0)),
            pl.BlockSpec((D, D), lambda i: (0, 0)),
            pl.BlockSpec((1, D), lambda i: (0, 0)),
        ],
        out_specs=pl.BlockSpec((BN, D), lambda i: (i, 0)),
        out_shape=jax.ShapeDtypeStruct((N, D), jnp.float32),
    )


def kernel(h, W1, b1, W2, b2, W3, b3, edge_index):
    src = edge_index[0].astype(jnp.int32).reshape(NW, RPT, K)
    dst = edge_index[1].astype(jnp.int32).reshape(NW, RPT, K)
    dstd = edge_index[1].astype(jnp.int32).reshape(NW, DRPT, DK)

    deg_k = _make_sc_deg()
    agg = _make_sc_agg()
    blk_res = _make_tc_block(True)
    blk_last = _make_tc_block(False)

    degp = deg_k(dstd)

    part = agg(h, src, dst)
    h1 = blk_res(part, degp, h, W1, b1.reshape(1, D))
    part = agg(h1, src, dst)
    h2 = blk_res(part, degp, h1, W2, b2.reshape(1, D))
    part = agg(h2, src, dst)
    h3 = blk_last(part, degp, h2, W3, b3.reshape(1, D))
    return h3


# speedup vs baseline: 1.3500x; 1.1107x over previous
import functools

import jax
import jax.numpy as jnp
from jax import lax
from jax.experimental import pallas as pl
from jax.experimental.pallas import tpu as pltpu
from jax.experimental.pallas import tpu_sc as plsc

N = 10000
NP = 10240
E = 320000
D = 128

NC = 2
NS = 16
NW = NC * NS
K = 50
ROWS = E // K
RPT = ROWS // NW
DK = 80
DROWS = E // DK
DRPT = DROWS // NW
NPT = NP // NS
DEGW = 16

_SC_PARAMS = pltpu.CompilerParams(use_tc_tiling_on_sc=False)
_MESH = dict(core_axis_name="c", subcore_axis_name="s")


def _zero_2d(ref, nrows, ncols):
    def row(r, carry):
        for j in range(ncols // 16):
            ref[r, pl.ds(j * 16, 16)] = jnp.zeros((16,), jnp.float32)
        return carry
    lax.fori_loop(0, nrows, row, 0)


def _sc_agg_body(h_hbm, src_hbm, dst_hbm, part_hbm, srcbuf, dstbuf, rows0,
                 rows1, rows2, rows3, acc, gsem0, gsem1, gsem2, gsem3, ssem0):
    bufs = (rows0, rows1, rows2, rows3)
    gsem = (gsem0, gsem1, gsem2, gsem3)
    NB = 4

    c = lax.axis_index("c")
    s = lax.axis_index("s")
    wid = c * NS + s

    _zero_2d(rows0, K, D)
    pltpu.async_copy(src_hbm.at[wid], srcbuf, gsem0)
    pltpu.async_copy(dst_hbm.at[wid], dstbuf, gsem1)
    for r in range(NPT // 40):
        pltpu.async_copy(rows0.at[pl.ds(0, 40)],
                         acc.at[pl.ds(s * NPT + r * 40, 40)], ssem0)
    pltpu.make_async_copy(src_hbm.at[wid], srcbuf, gsem0).wait()
    pltpu.make_async_copy(dst_hbm.at[wid], dstbuf, gsem1).wait()
    for r in range(NPT // 40):
        pltpu.make_async_copy(rows0.at[pl.ds(0, 40)],
                              acc.at[pl.ds(s * NPT, 40)], ssem0).wait()
    plsc.subcore_barrier()

    for b in range(NB):
        pltpu.async_copy(h_hbm.at[srcbuf.at[b]], bufs[b], gsem[b])

    def step(i, carry):
        for b in range(NB):
            j = i * NB + b
            pltpu.make_async_copy(h_hbm.at[srcbuf.at[j]], bufs[b],
                                  gsem[b]).wait()
            pltpu.sync_copy(bufs[b], acc.at[dstbuf.at[j]], add=True)

            @pl.when(j + NB < RPT)
            def _():
                pltpu.async_copy(h_hbm.at[srcbuf.at[j + NB]], bufs[b],
                                 gsem[b])
        return carry
    lax.fori_loop(0, RPT // NB, step, 0)

    plsc.subcore_barrier()

    pltpu.sync_copy(acc.at[pl.ds(s * NPT, NPT)],
                    part_hbm.at[c, pl.ds(s * NPT, NPT)])


def _make_sc_agg():
    return pl.kernel(
        _sc_agg_body,
        out_type=jax.ShapeDtypeStruct((NC, NP, D), jnp.float32),
        mesh=plsc.VectorSubcoreMesh(**_MESH),
        scratch_types=(
            pltpu.VMEM((RPT, K), jnp.int32),
            pltpu.VMEM((RPT, K), jnp.int32),
            pltpu.VMEM((K, D), jnp.float32),
            pltpu.VMEM((K, D), jnp.float32),
            pltpu.VMEM((K, D), jnp.float32),
            pltpu.VMEM((K, D), jnp.float32),
            pltpu.VMEM_SHARED((NP, D), jnp.float32),
            pltpu.SemaphoreType.DMA,
            pltpu.SemaphoreType.DMA,
            pltpu.SemaphoreType.DMA,
            pltpu.SemaphoreType.DMA,
            pltpu.SemaphoreType.DMA,
        ),
        compiler_params=_SC_PARAMS,
    )


def _sc_deg_body(dst_hbm, degp_hbm, dstbuf, onesb, degacc, sem):
    c = lax.axis_index("c")
    s = lax.axis_index("s")
    wid = c * NS + s

    pltpu.sync_copy(dst_hbm.at[wid], dstbuf)

    _zero_2d(onesb, DK, DEGW)
    for r in range(NPT // DK):
        pltpu.sync_copy(onesb, degacc.at[pl.ds(s * NPT + r * DK, DK)])

    def orow(r, carry):
        onesb[r, pl.ds(0, 16)] = jnp.ones((16,), jnp.float32)
        return carry
    lax.fori_loop(0, DK, orow, 0)
    plsc.subcore_barrier()

    def fire(j, carry):
        pltpu.async_copy(onesb, degacc.at[dstbuf.at[j]], sem, add=True)
        return carry
    lax.fori_loop(0, DRPT, fire, 0)

    def drain(j, carry):
        pltpu.make_async_copy(onesb, degacc.at[dstbuf.at[0]], sem).wait()
        return carry
    lax.fori_loop(0, DRPT, drain, 0)

    plsc.subcore_barrier()
    pltpu.sync_copy(degacc.at[pl.ds(s * NPT, NPT)],
                    degp_hbm.at[c, pl.ds(s * NPT, NPT)])


def _make_sc_deg():
    return pl.kernel(
        _sc_deg_body,
        out_type=jax.ShapeDtypeStruct((NC, NP, DEGW), jnp.float32),
        mesh=plsc.VectorSubcoreMesh(**_MESH),
        scratch_types=(
            pltpu.VMEM((DRPT, DK), jnp.int32),
            pltpu.VMEM((DK, DEGW), jnp.float32),
            pltpu.VMEM_SHARED((NP, DEGW), jnp.float32),
            pltpu.SemaphoreType.DMA,
        ),
        compiler_params=_SC_PARAMS,
    )


def _tc_block_body(residual, p_ref, degp_ref, h_ref, w_ref, b_ref, o_ref):
    deg = degp_ref[0, :, 0:1] + degp_ref[1, :, 0:1]
    agg = (p_ref[0] + p_ref[1]) / jnp.maximum(deg, 1.0)
    y = jnp.dot(agg, w_ref[...], preferred_element_type=jnp.float32)
    y = jnp.maximum(y + b_ref[...], 0.0)
    if residual:
        y = y + h_ref[...]
    o_ref[...] = y


def _make_tc_block(residual):
    BN = 1000
    return pl.pallas_call(
        functools.partial(_tc_block_body, residual),
        grid=(N // BN,),
        in_specs=[
            pl.BlockSpec((NC, BN, D), lambda i: (0, i, 0)),
            pl.BlockSpec((NC, BN, DEGW), lambda i: (0, i, 0)),
            pl.BlockSpec((BN, D), lambda i: (i, 0)),
            pl.BlockSpec((D, D), lambda i: (0, 0)),
            pl.BlockSpec((1, D), lambda i: (0, 0)),
        ],
        out_specs=pl.BlockSpec((BN, D), lambda i: (i, 0)),
        out_shape=jax.ShapeDtypeStruct((N, D), jnp.float32),
    )


def kernel(h, W1, b1, W2, b2, W3, b3, edge_index):
    src = edge_index[0].astype(jnp.int32).reshape(NW, RPT, K)
    dst = edge_index[1].astype(jnp.int32).reshape(NW, RPT, K)
    dstd = edge_index[1].astype(jnp.int32).reshape(NW, DRPT, DK)

    deg_k = _make_sc_deg()
    agg = _make_sc_agg()
    blk_res = _make_tc_block(True)
    blk_last = _make_tc_block(False)

    degp = deg_k(dstd)

    part = agg(h, src, dst)
    h1 = blk_res(part, degp, h, W1, b1.reshape(1, D))
    part = agg(h1, src, dst)
    h2 = blk_res(part, degp, h1, W2, b2.reshape(1, D))
    part = agg(h2, src, dst)
    h3 = blk_last(part, degp, h2, W3, b3.reshape(1, D))
    return h3


# speedup vs baseline: 1.4028x; 1.0391x over previous
import functools

import jax
import jax.numpy as jnp
from jax import lax
from jax.experimental import pallas as pl
from jax.experimental.pallas import tpu as pltpu
from jax.experimental.pallas import tpu_sc as plsc

N = 10000
NP = 10240
E = 320000
D = 128

NC = 2
NS = 16
NW = NC * NS
K = 40
ROWS = E // K
RPT = ROWS // NW
DK = 80
DROWS = E // DK
DRPT = DROWS // NW
NPT = NP // NS
DEGW = 16

_SC_PARAMS = pltpu.CompilerParams(use_tc_tiling_on_sc=False)
_MESH = dict(core_axis_name="c", subcore_axis_name="s")


def _zero_2d(ref, nrows, ncols):
    def row(r, carry):
        for j in range(ncols // 16):
            ref[r, pl.ds(j * 16, 16)] = jnp.zeros((16,), jnp.float32)
        return carry
    lax.fori_loop(0, nrows, row, 0)


def _sc_agg_body(h_hbm, src_hbm, dst_hbm, part_hbm, srcbuf, dstbuf, rows_all,
                 acc, gsem0, gsem1, gsem2, gsem3, gsem4, ssem0):
    NB = 5
    bufs = [rows_all.at[pl.ds(b * K, K)] for b in range(NB)]
    gsem = (gsem0, gsem1, gsem2, gsem3, gsem4)
    rows0 = bufs[0]

    c = lax.axis_index("c")
    s = lax.axis_index("s")
    wid = c * NS + s

    _zero_2d(rows0, K, D)
    pltpu.async_copy(src_hbm.at[wid], srcbuf, gsem0)
    pltpu.async_copy(dst_hbm.at[wid], dstbuf, gsem1)
    for r in range(NPT // 40):
        pltpu.async_copy(rows0.at[pl.ds(0, 40)],
                         acc.at[pl.ds(s * NPT + r * 40, 40)], ssem0)
    pltpu.make_async_copy(src_hbm.at[wid], srcbuf, gsem0).wait()
    pltpu.make_async_copy(dst_hbm.at[wid], dstbuf, gsem1).wait()
    for r in range(NPT // 40):
        pltpu.make_async_copy(rows0.at[pl.ds(0, 40)],
                              acc.at[pl.ds(s * NPT, 40)], ssem0).wait()
    plsc.subcore_barrier()

    for b in range(NB):
        pltpu.async_copy(h_hbm.at[srcbuf.at[b]], bufs[b], gsem[b])

    def step(i, carry):
        for b in range(NB):
            j = i * NB + b
            pltpu.make_async_copy(h_hbm.at[srcbuf.at[j]], bufs[b],
                                  gsem[b]).wait()
            pltpu.sync_copy(bufs[b], acc.at[dstbuf.at[j]], add=True)

            @pl.when(j + NB < RPT)
            def _():
                pltpu.async_copy(h_hbm.at[srcbuf.at[j + NB]], bufs[b],
                                 gsem[b])
        return carry
    lax.fori_loop(0, RPT // NB, step, 0)

    plsc.subcore_barrier()

    pltpu.sync_copy(acc.at[pl.ds(s * NPT, NPT)],
                    part_hbm.at[c, pl.ds(s * NPT, NPT)])


def _make_sc_agg():
    return pl.kernel(
        _sc_agg_body,
        out_type=jax.ShapeDtypeStruct((NC, NP, D), jnp.float32),
        mesh=plsc.VectorSubcoreMesh(**_MESH),
        scratch_types=(
            pltpu.VMEM((RPT, K), jnp.int32),
            pltpu.VMEM((RPT, K), jnp.int32),
            pltpu.VMEM((5 * K, D), jnp.float32),
            pltpu.VMEM_SHARED((NP, D), jnp.float32),
        ) + tuple(pltpu.SemaphoreType.DMA for _ in range(6)),
        compiler_params=_SC_PARAMS,
    )


def _sc_deg_body(dst_hbm, degp_hbm, dstbuf, onesb, degacc, sem):
    c = lax.axis_index("c")
    s = lax.axis_index("s")
    wid = c * NS + s

    pltpu.sync_copy(dst_hbm.at[wid], dstbuf)

    _zero_2d(onesb, DK, DEGW)
    for r in range(NPT // DK):
        pltpu.sync_copy(onesb, degacc.at[pl.ds(s * NPT + r * DK, DK)])

    def orow(r, carry):
        onesb[r, pl.ds(0, 16)] = jnp.ones((16,), jnp.float32)
        return carry
    lax.fori_loop(0, DK, orow, 0)
    plsc.subcore_barrier()

    def fire(j, carry):
        pltpu.async_copy(onesb, degacc.at[dstbuf.at[j]], sem, add=True)
        return carry
    lax.fori_loop(0, DRPT, fire, 0)

    def drain(j, carry):
        pltpu.make_async_copy(onesb, degacc.at[dstbuf.at[0]], sem).wait()
        return carry
    lax.fori_loop(0, DRPT, drain, 0)

    plsc.subcore_barrier()
    pltpu.sync_copy(degacc.at[pl.ds(s * NPT, NPT)],
                    degp_hbm.at[c, pl.ds(s * NPT, NPT)])


def _make_sc_deg():
    return pl.kernel(
        _sc_deg_body,
        out_type=jax.ShapeDtypeStruct((NC, NP, DEGW), jnp.float32),
        mesh=plsc.VectorSubcoreMesh(**_MESH),
        scratch_types=(
            pltpu.VMEM((DRPT, DK), jnp.int32),
            pltpu.VMEM((DK, DEGW), jnp.float32),
            pltpu.VMEM_SHARED((NP, DEGW), jnp.float32),
            pltpu.SemaphoreType.DMA,
        ),
        compiler_params=_SC_PARAMS,
    )


def _tc_block_body(residual, p_ref, degp_ref, h_ref, w_ref, b_ref, o_ref):
    deg = degp_ref[0, :, 0:1] + degp_ref[1, :, 0:1]
    agg = (p_ref[0] + p_ref[1]) / jnp.maximum(deg, 1.0)
    y = jnp.dot(agg, w_ref[...], preferred_element_type=jnp.float32)
    y = jnp.maximum(y + b_ref[...], 0.0)
    if residual:
        y = y + h_ref[...]
    o_ref[...] = y


def _make_tc_block(residual):
    BN = 1000
    return pl.pallas_call(
        functools.partial(_tc_block_body, residual),
        grid=(N // BN,),
        in_specs=[
            pl.BlockSpec((NC, BN, D), lambda i: (0, i, 0)),
            pl.BlockSpec((NC, BN, DEGW), lambda i: (0, i, 0)),
            pl.BlockSpec((BN, D), lambda i: (i, 0)),
            pl.BlockSpec((D, D), lambda i: (0, 0)),
            pl.BlockSpec((1, D), lambda i: (0, 0)),
        ],
        out_specs=pl.BlockSpec((BN, D), lambda i: (i, 0)),
        out_shape=jax.ShapeDtypeStruct((N, D), jnp.float32),
    )


def kernel(h, W1, b1, W2, b2, W3, b3, edge_index):
    src = edge_index[0].astype(jnp.int32).reshape(NW, RPT, K)
    dst = edge_index[1].astype(jnp.int32).reshape(NW, RPT, K)
    dstd = edge_index[1].astype(jnp.int32).reshape(NW, DRPT, DK)

    deg_k = _make_sc_deg()
    agg = _make_sc_agg()
    blk_res = _make_tc_block(True)
    blk_last = _make_tc_block(False)

    degp = deg_k(dstd)

    part = agg(h, src, dst)
    h1 = blk_res(part, degp, h, W1, b1.reshape(1, D))
    part = agg(h1, src, dst)
    h2 = blk_res(part, degp, h1, W2, b2.reshape(1, D))
    part = agg(h2, src, dst)
    h3 = blk_last(part, degp, h2, W3, b3.reshape(1, D))
    return h3
